# Initial kernel scaffold; baseline (speedup 1.0000x reference)
import jax
import jax.numpy as jnp
from jax.experimental import pallas as pl


def kernel(points, w1a, w1c, b1, w2a, w2c, b2, w3a, w3c, b3, w4a, w4c, b4, w5_1, w5_2, w5_3, w5_4, b5, w6, b6, w7, b7, w8, b8):
    raise NotImplementedError("write your pallas kernel here")


# speedup vs baseline: 1.2639x; 1.2639x over previous
import functools

import jax
import jax.numpy as jnp
from jax.experimental import pallas as pl
from jax.experimental.pallas import tpu as pltpu

_LEAKY_SLOPE = 0.2


def _leaky_relu(x):
    return jnp.where(x >= 0, x, _LEAKY_SLOPE * x)


def _edge_conv_layer(x, wa, wc, bias, k):
    n = x.shape[0]
    pa = jnp.dot(x, wa, preferred_element_type=jnp.float32)
    pc = jnp.dot(x, wc, preferred_element_type=jnp.float32) + bias

    g = jnp.dot(x, x.T, preferred_element_type=jnp.float32)
    sq = jnp.sum(x * x, axis=-1, keepdims=True)
    dT = 2.0 * g - sq

    nbits = max(1, (n - 1).bit_length())
    diT = pltpu.bitcast(dT, jnp.int32)
    fkeyT = jnp.where(diT >= 0, diT, diT ^ jnp.int32(0x7FFFFFFF))
    nbr = jax.lax.broadcasted_iota(jnp.int32, (n, n), 0)
    keyT = (fkeyT & jnp.int32(-(1 << nbits))) | (jnp.int32(n - 1) - nbr)

    pa_b = pa.astype(jnp.bfloat16)
    int_min = jnp.int32(-(2 ** 31))
    maxnbr = None
    for _ in range(k):
        m = jnp.max(keyT, axis=0, keepdims=True)
        ohT = keyT >= m
        sel = jnp.where(ohT, 1.0, 0.0).astype(jnp.bfloat16)
        keyT = jnp.where(ohT, int_min, keyT)
        gat = jax.lax.dot_general(
            sel, pa_b, (((0,), (0,)), ((), ())),
            preferred_element_type=jnp.float32)
        maxnbr = gat if maxnbr is None else jnp.maximum(maxnbr, gat)
    return _leaky_relu(maxnbr + pc)


def _trunk_kernel(x_ref,
                  w1a, w1c, b1, w2a, w2c, b2, w3a, w3c, b3, w4a, w4c, b4,
                  w5_1, w5_2, w5_3, w5_4, b5,
                  out_ref, *, k):
    x0 = x_ref[0]
    x1 = _edge_conv_layer(x0, w1a[...], w1c[...], b1[...], k)
    x2 = _edge_conv_layer(x1, w2a[...], w2c[...], b2[...], k)
    x3 = _edge_conv_layer(x2, w3a[...], w3c[...], b3[...], k)
    x4 = _edge_conv_layer(x3, w4a[...], w4c[...], b4[...], k)
    y = (jnp.dot(x1.astype(jnp.bfloat16), w5_1[...], preferred_element_type=jnp.float32)
         + jnp.dot(x2.astype(jnp.bfloat16), w5_2[...], preferred_element_type=jnp.float32)
         + jnp.dot(x3.astype(jnp.bfloat16), w5_3[...], preferred_element_type=jnp.float32)
         + jnp.dot(x4.astype(jnp.bfloat16), w5_4[...], preferred_element_type=jnp.float32)
         + b5[...])
    y = _leaky_relu(y)
    out_ref[0] = jnp.max(y, axis=0, keepdims=True)


_TRUNK_WEIGHT_NAMES = (
    'w1a', 'w1c', 'b1', 'w2a', 'w2c', 'b2', 'w3a', 'w3c', 'b3',
    'w4a', 'w4c', 'b4', 'w5_1', 'w5_2', 'w5_3', 'w5_4', 'b5')


def _trunk_pooled(x, params, k):
    B, N, _ = x.shape
    const_spec = lambda arr: pl.BlockSpec(arr.shape, lambda b: (0, 0))
    in_specs = [pl.BlockSpec((1, N, 3), lambda b: (b, 0, 0))]
    operands = [x]
    for name in _TRUNK_WEIGHT_NAMES:
        arr = params[name]
        in_specs.append(const_spec(arr))
        operands.append(arr)
    return pl.pallas_call(
        functools.partial(_trunk_kernel, k=k),
        out_shape=jax.ShapeDtypeStruct((B, 1, 1024), jnp.float32),
        grid=(B,),
        in_specs=in_specs,
        out_specs=pl.BlockSpec((1, 1, 1024), lambda b: (b, 0, 0)),
        compiler_params=pltpu.CompilerParams(
            dimension_semantics=("parallel",)),
    )(*operands)


def _head_kernel(pooled_ref, w6, b6, w7, b7, w8, b8, out_ref):
    p = pooled_ref[...]
    h = jnp.maximum(
        jnp.dot(p.astype(jnp.bfloat16), w6[...],
                preferred_element_type=jnp.float32) + b6[...], 0.0)
    h = jnp.maximum(
        jnp.dot(h.astype(jnp.bfloat16), w7[...],
                preferred_element_type=jnp.float32) + b7[...], 0.0)
    out_ref[...] = jnp.tanh(
        jnp.dot(h.astype(jnp.bfloat16), w8[...],
                preferred_element_type=jnp.float32) + b8[...])


def _head_mlp(pooled, params):
    B = pooled.shape[0]
    Cout = params['w8'].shape[1]
    BT = 256 if B % 256 == 0 else B
    const_spec = lambda arr: pl.BlockSpec(arr.shape, lambda b: (0, 0))
    return pl.pallas_call(
        _head_kernel,
        out_shape=jax.ShapeDtypeStruct((B, Cout), jnp.float32),
        grid=(B // BT,),
        in_specs=[pl.BlockSpec((BT, 1024), lambda b: (b, 0))]
                 + [const_spec(params[nm]) for nm in
                    ('w6', 'b6', 'w7', 'b7', 'w8', 'b8')],
        out_specs=pl.BlockSpec((BT, Cout), lambda b: (b, 0)),
        compiler_params=pltpu.CompilerParams(
            dimension_semantics=("parallel",)),
    )(pooled, params['w6'], params['b6'], params['w7'], params['b7'],
      params['w8'], params['b8'])


def kernel(points, w1a, w1c, b1, w2a, w2c, b2, w3a, w3c, b3, w4a, w4c, b4,
           w5_1, w5_2, w5_3, w5_4, b5, w6, b6, w7, b7, w8, b8):
    params = {
        'w1a': w1a, 'w1c': w1c, 'b1': b1,
        'w2a': w2a, 'w2c': w2c, 'b2': b2,
        'w3a': w3a, 'w3c': w3c, 'b3': b3,
        'w4a': w4a, 'w4c': w4c, 'b4': b4,
        'w5_1': w5_1, 'w5_2': w5_2, 'w5_3': w5_3, 'w5_4': w5_4, 'b5': b5,
        'w6': w6, 'b6': b6, 'w7': w7, 'b7': b7, 'w8': w8, 'b8': b8,
    }
    x = jnp.transpose(points, (0, 2, 1)).astype(jnp.float32)
    B = x.shape[0]
    num_cp = 8
    pooled = _trunk_pooled(x, params, 20).reshape(B, 1024)
    out = _head_mlp(pooled, params)
    return out.reshape(B, num_cp * num_cp, 3)
